# baseline (speedup 1.0000x reference)
import functools

import jax
import jax.numpy as jnp
from jax import lax
from jax.experimental import pallas as pl
from jax.experimental.pallas import tpu as pltpu
from jax.experimental.pallas import tpu_sc as plsc

_NB = 4


def _build(B, S, V, D):
    NC, NS = 2, 16
    NW = NC * NS
    SP = S + (-S) % 4
    CW = 2 * SP
    assert CW <= 128 and D == 32 and B % (2 * NW * _NB) == 0
    CPW = B // (2 * NW)
    RPW = B // NW
    scale = 1.0 / B

    mesh = plsc.VectorSubcoreMesh(core_axis_name="c", subcore_axis_name="s")

    @functools.partial(
        pl.kernel,
        mesh=mesh,
        out_type=jax.ShapeDtypeStruct((B, D), jnp.float32),
        scratch_types=[
            pltpu.VMEM((CPW, CW), jnp.int32),
            [pltpu.VMEM((CW, D), jnp.float32)] * _NB,
            pltpu.VMEM((RPW, D), jnp.float32),
            [pltpu.SemaphoreType.DMA] * _NB,
        ],
        compiler_params=pltpu.CompilerParams(use_tc_tiling_on_sc=False),
    )
    def k(doc_hbm, table_hbm, out_hbm, idx_v, rows, out_v, sems):
        wid = lax.axis_index("s") * NC + lax.axis_index("c")
        pltpu.sync_copy(doc_hbm.at[pl.ds(wid * CPW, CPW)], idx_v)

        for b in range(_NB):
            pltpu.async_copy(table_hbm.at[idx_v.at[b]], rows[b], sems[b])

        def body(i, _):
            j0 = i * _NB
            for b in range(_NB):
                j = j0 + b
                rv = rows[b]
                pltpu.make_async_copy(
                    table_hbm.at[idx_v.at[j]], rv, sems[b]).wait()
                for h in range(2):
                    base = h * SP
                    ev = [None, None]
                    od = [None, None]
                    for t in range(S):
                        tgt = ev if t % 2 == 0 else od
                        for d in range(2):
                            v = rv[base + t, pl.ds(16 * d, 16)]
                            tgt[d] = v if tgt[d] is None else tgt[d] + v
                    out_v[2 * j + h, pl.ds(0, 16)] = (ev[0] + od[0]) * scale
                    out_v[2 * j + h, pl.ds(16, 16)] = (ev[1] + od[1]) * scale

                nj = j + _NB

                @pl.when(nj < CPW)
                def _():
                    pltpu.async_copy(table_hbm.at[idx_v.at[nj]], rv, sems[b])

            return 0

        lax.fori_loop(0, CPW // _NB, body, 0)
        pltpu.sync_copy(out_v, out_hbm.at[pl.ds(wid * RPW, RPW)])

    return k


def kernel(document, table):
    B, S = document.shape
    V, D = table.shape
    SP = S + (-S) % 4
    doc_p = jnp.pad(document, ((0, 0), (0, SP - S)))
    doc2 = doc_p.reshape(B // 2, 2 * SP)
    return _build(B, S, V, D)(doc2, table)
